# baseline (speedup 1.0000x reference)
import functools

import jax
import jax.numpy as jnp
from jax.experimental import pallas as pl

_H = 64
_G = 64
_EPS = 1e-5


def _row(v):
    return v.reshape(1, -1)


def _ln_in(x, g, b):
    m = jnp.mean(x, axis=-1, keepdims=True)
    v = jnp.mean((x - m) ** 2, axis=-1, keepdims=True)
    return (x - m) * jax.lax.rsqrt(v + _EPS) * g + b


def _dot(a, b):
    return jnp.dot(a, b, preferred_element_type=jnp.float32,
                   precision=jax.lax.Precision.HIGHEST)


def _mlp_tail(acc, b0, w1, b1, w2, b2, g, be, w3, b3):
    x = jnp.maximum(acc + b0, 0.0)
    x = jnp.maximum(_dot(x, w1) + b1, 0.0)
    x = jnp.maximum(_dot(x, w2) + b2, 0.0)
    x = _ln_in(x, g, be)
    return _dot(x, w3) + b3


def _mlp_weights(p, splits):
    w0s = []
    off = 0
    for d in splits:
        w0s.append(p['W0'][off:off + d, :])
        off += d
    tail = [_row(p['b0']), p['W1'], _row(p['b1']), p['W2'], _row(p['b2']),
            _row(p['g']), _row(p['be']), p['W3'], _row(p['b3'])]
    return w0s, tail


def _fused_mlp(parts, w0s, tail, idx=None, oh_mat=None, res=None, block=None):
    rows = parts[0].shape[0]
    br = block
    grid = rows // br
    nparts = len(parts)
    has_idx = idx is not None
    has_res = res is not None

    def body(*refs):
        it = iter(refs)
        p_refs = [next(it) for _ in range(nparts)]
        idx_ref = next(it) if has_idx else None
        res_ref = next(it) if has_res else None
        w0_refs = [next(it) for _ in range(nparts)]
        oh_ref = next(it) if has_idx else None
        b0, w1, b1, w2, b2, g, be, w3, b3 = (next(it) for _ in range(9))
        out_ref = next(it)
        acc = _dot(p_refs[0][...], w0_refs[0][...])
        for pr, wr in zip(p_refs[1:], w0_refs[1:]):
            acc = acc + _dot(pr[...], wr[...])
        if has_idx:
            lanes = jax.lax.broadcasted_iota(jnp.int32, (1, _G), 1)
            oh = (idx_ref[...] == lanes).astype(jnp.float32)
            acc = acc + _dot(oh, oh_ref[...])
        out = _mlp_tail(acc, b0[...], w1[...], b1[...], w2[...], b2[...],
                        g[...], be[...], w3[...], b3[...])
        if has_res:
            out = out + res_ref[...]
        out_ref[...] = out

    inputs = list(parts)
    in_specs = [pl.BlockSpec((br, p.shape[1]), lambda i: (i, 0)) for p in parts]
    if has_idx:
        inputs.append(idx)
        in_specs.append(pl.BlockSpec((br, 1), lambda i: (i, 0)))
    if has_res:
        inputs.append(res)
        in_specs.append(pl.BlockSpec((br, res.shape[1]), lambda i: (i, 0)))
    full = lambda a: pl.BlockSpec(a.shape, lambda i: (0,) * a.ndim)
    weights = list(w0s) + ([oh_mat] if has_idx else []) + tail
    inputs += weights
    in_specs += [full(w) for w in weights]
    out_dim = tail[-2].shape[1]
    return pl.pallas_call(
        body,
        grid=(grid,),
        in_specs=in_specs,
        out_specs=pl.BlockSpec((br, out_dim), lambda i: (i, 0)),
        out_shape=jax.ShapeDtypeStruct((rows, out_dim), jnp.float32),
    )(*inputs)


def _edge_stage(h_row, h_col, e, bidx, cp, u, block):
    rows = h_row.shape[0]
    br = block
    grid = rows // br
    ew0, etail = _mlp_weights(cp['edge'], [_H, _H, _H])
    pe = u @ cp['edge']['W0'][3 * _H:, :]
    nw0, ntail = _mlp_weights(cp['node1'], [_H, _H])

    def body(hr_ref, hc_ref, e_ref, b_ref,
             ewa, ewb, ewc, pe_ref, eb0, ew1, eb1, ew2, eb2, eg, ebe, ew3, eb3,
             nwa, nwb, nb0, nw1, nb1, nw2, nb2, ng, nbe, nw3, nb3,
             e_out, m_out):
        hr = hr_ref[...]
        lanes = jax.lax.broadcasted_iota(jnp.int32, (1, _G), 1)
        oh = (b_ref[...] == lanes).astype(jnp.float32)
        acc = (_dot(hr, ewa[...]) + _dot(hc_ref[...], ewb[...]) +
               _dot(e_ref[...], ewc[...]) + _dot(oh, pe_ref[...]))
        e_new = e_ref[...] + _mlp_tail(acc, eb0[...], ew1[...], eb1[...],
                                       ew2[...], eb2[...], eg[...], ebe[...],
                                       ew3[...], eb3[...])
        e_out[...] = e_new
        acc2 = _dot(hr, nwa[...]) + _dot(e_new, nwb[...])
        m_out[...] = _mlp_tail(acc2, nb0[...], nw1[...], nb1[...], nw2[...],
                               nb2[...], ng[...], nbe[...], nw3[...], nb3[...])

    full = lambda a: pl.BlockSpec(a.shape, lambda i: (0,) * a.ndim)
    weights = ew0 + [pe] + etail + nw0 + ntail
    in_specs = ([pl.BlockSpec((br, _H), lambda i: (i, 0))] * 3 +
                [pl.BlockSpec((br, 1), lambda i: (i, 0))] +
                [full(w) for w in weights])
    return pl.pallas_call(
        body,
        grid=(grid,),
        in_specs=in_specs,
        out_specs=[pl.BlockSpec((br, _H), lambda i: (i, 0))] * 2,
        out_shape=[jax.ShapeDtypeStruct((rows, _H), jnp.float32)] * 2,
    )(h_row, h_col, e, bidx, *weights)


def _node_stage(h, agg, batch2, cp, u, block):
    rows = h.shape[0]
    br = block
    grid = rows // br
    w0, tail = _mlp_weights(cp['node2'], [_H, _H])
    pn = u @ cp['node2']['W0'][2 * _H:, :]

    def body(h_ref, a_ref, b_ref, wa, wb, pn_ref,
             b0, w1, b1, w2, b2, g, be, w3, b3,
             h_out, s_out, sq_out, c_out, mn_out, mx_out):
        bcol = b_ref[...]
        lanes = jax.lax.broadcasted_iota(jnp.int32, (1, _G), 1)
        oh = (bcol == lanes).astype(jnp.float32)
        acc = _dot(h_ref[...], wa[...]) + _dot(a_ref[...], wb[...]) + _dot(oh, pn_ref[...])
        hn = h_ref[...] + _mlp_tail(acc, b0[...], w1[...], b1[...], w2[...],
                                    b2[...], g[...], be[...], w3[...], b3[...])
        h_out[...] = hn

        @pl.when(pl.program_id(0) == 0)
        def _init():
            s_out[...] = jnp.zeros_like(s_out)
            sq_out[...] = jnp.zeros_like(sq_out)
            c_out[...] = jnp.zeros_like(c_out)
            mn_out[...] = jnp.full_like(mn_out, jnp.inf)
            mx_out[...] = jnp.full_like(mx_out, -jnp.inf)

        contract = (((0,), (0,)), ((), ()))
        hi_prec = jax.lax.Precision.HIGHEST
        s_out[...] += jax.lax.dot_general(oh, hn, contract,
                                          preferred_element_type=jnp.float32,
                                          precision=hi_prec)
        sq_out[...] += jax.lax.dot_general(oh, hn * hn, contract,
                                           preferred_element_type=jnp.float32,
                                           precision=hi_prec)
        ones = jnp.ones((br, 1), jnp.float32)
        c_out[...] += jax.lax.dot_general(oh, ones, contract,
                                          preferred_element_type=jnp.float32,
                                          precision=hi_prec)

        def upd(gi, _):
            sel = bcol == gi
            lo = jnp.min(jnp.where(sel, hn, jnp.inf), axis=0, keepdims=True)
            hi = jnp.max(jnp.where(sel, hn, -jnp.inf), axis=0, keepdims=True)
            mn_out[pl.ds(gi, 1), :] = jnp.minimum(mn_out[pl.ds(gi, 1), :], lo)
            mx_out[pl.ds(gi, 1), :] = jnp.maximum(mx_out[pl.ds(gi, 1), :], hi)
            return 0

        jax.lax.fori_loop(0, _G, upd, 0)

    full = lambda a: pl.BlockSpec(a.shape, lambda i: (0,) * a.ndim)
    weights = w0 + [pn] + tail
    in_specs = ([pl.BlockSpec((br, _H), lambda i: (i, 0))] * 2 +
                [pl.BlockSpec((br, 1), lambda i: (i, 0))] +
                [full(w) for w in weights])
    acc_spec = pl.BlockSpec((_G, _H), lambda i: (0, 0))
    cnt_spec = pl.BlockSpec((_G, 1), lambda i: (0, 0))
    return pl.pallas_call(
        body,
        grid=(grid,),
        in_specs=in_specs,
        out_specs=[pl.BlockSpec((br, _H), lambda i: (i, 0)),
                   acc_spec, acc_spec, cnt_spec, acc_spec, acc_spec],
        out_shape=[jax.ShapeDtypeStruct((rows, _H), jnp.float32),
                   jax.ShapeDtypeStruct((_G, _H), jnp.float32),
                   jax.ShapeDtypeStruct((_G, _H), jnp.float32),
                   jax.ShapeDtypeStruct((_G, 1), jnp.float32),
                   jax.ShapeDtypeStruct((_G, _H), jnp.float32),
                   jax.ShapeDtypeStruct((_G, _H), jnp.float32)],
    )(h, agg, batch2, *weights)


def _glob_stage(u, s, sq, cnt, mn, mx, cp):
    w0, tail = _mlp_weights(cp['glob'], [_H] * 5)

    def body(u_ref, s_ref, sq_ref, c_ref, mn_ref, mx_ref,
             wa, wb, wc, wd, we, b0, w1, b1, w2, b2, g, be, w3, b3, out_ref):
        c = jnp.maximum(c_ref[...], 1.0)
        me = s_ref[...] / c
        std = sq_ref[...] / c - me * me
        acc = (_dot(u_ref[...], wa[...]) + _dot(s_ref[...], wb[...]) +
               _dot(mn_ref[...], wc[...]) + _dot(mx_ref[...], wd[...]) +
               _dot(std, we[...]))
        out_ref[...] = u_ref[...] + _mlp_tail(
            acc, b0[...], w1[...], b1[...], w2[...], b2[...], g[...], be[...],
            w3[...], b3[...])

    full = lambda a: pl.BlockSpec(a.shape, lambda i: (0,) * a.ndim)
    weights = w0 + tail
    in_specs = [full(a) for a in (u, s, sq, cnt, mn, mx)] + [full(w) for w in weights]
    return pl.pallas_call(
        body,
        grid=(1,),
        in_specs=in_specs,
        out_specs=pl.BlockSpec((_G, _H), lambda i: (0, 0)),
        out_shape=jax.ShapeDtypeStruct((_G, _H), jnp.float32),
    )(u, s, sq, cnt, mn, mx, *weights)


def _decoders(pooled, decs):
    def body(p_ref, *refs):
        out_ref = refs[-1]
        cols = []
        for d in range(2):
            g0, be0, w0, bb0, g1, be1, w1, bb1 = refs[8 * d:8 * d + 8]
            x1 = _ln_in(p_ref[...], g0[...], be0[...])
            x1 = jnp.maximum(_dot(x1, w0[...]) + bb0[...], 0.0)
            x1 = _ln_in(x1, g1[...], be1[...])
            x1 = jnp.maximum(_dot(x1, w1[...]) + bb1[...], 0.0)
            cols.append(x1)
        out_ref[...] = jnp.concatenate(cols, axis=1)

    weights = []
    for dp in decs:
        weights += [_row(dp['g0']), _row(dp['be0']), dp['W0'], _row(dp['bb0']),
                    _row(dp['g1']), _row(dp['be1']), dp['W1'], _row(dp['bb1'])]
    full = lambda a: pl.BlockSpec(a.shape, lambda i: (0,) * a.ndim)
    return pl.pallas_call(
        body,
        grid=(1,),
        in_specs=[full(pooled)] + [full(w) for w in weights],
        out_specs=pl.BlockSpec((_G, 2), lambda i: (0, 0)),
        out_shape=jax.ShapeDtypeStruct((_G, 2), jnp.float32),
    )(pooled, *weights)


def kernel(x, edge_attr, params, edge_index, batch):
    n = x.shape[0]
    e_cnt = edge_attr.shape[0]
    row = edge_index[0]
    col = edge_index[1]
    bn = 2000
    be = 4000

    batch2 = batch.astype(jnp.int32).reshape(n, 1)
    bidx_e = jnp.take(batch.astype(jnp.int32), row, axis=0).reshape(e_cnt, 1)

    w0, tail = _mlp_weights(params['node_enc'], [x.shape[1]])
    h = _fused_mlp([x], w0, tail, block=bn)

    ef = x[:, jnp.array([0, 3])]
    e_in = jnp.concatenate(
        [edge_attr.reshape(-1, 1),
         jnp.take(ef, row, axis=0) - jnp.take(ef, col, axis=0)], axis=-1)
    w0, tail = _mlp_weights(params['edge_enc'], [3])
    e = _fused_mlp([e_in], w0, tail, block=be)

    u = jnp.zeros((_G, _H), jnp.float32)
    s = None
    for cp in params['convs']:
        h_row = jnp.take(h, row, axis=0)
        h_col = jnp.take(h, col, axis=0)
        e, msg = _edge_stage(h_row, h_col, e, bidx_e, cp, u, be)
        agg = jax.ops.segment_sum(msg, col, num_segments=n)
        h, s, sq, cnt, mn, mx = _node_stage(h, agg, batch2, cp, u, bn)
        u = _glob_stage(u, s, sq, cnt, mn, mx, cp)

    return _decoders(s, params['dec'])
